# baseline (speedup 1.0000x reference)
import functools

import jax
import jax.numpy as jnp
from jax import lax
from jax.experimental import pallas as pl
from jax.experimental.pallas import tpu as pltpu
from jax.experimental.pallas import tpu_sc as plsc

_VOCAB = 65536
_MD = 512
_H = 256
_NTOK = 16384 * 20

_RB = 1024

_NC, _NS, _LANES = 2, 16, 16
_NW = _NC * _NS
_BPW = _NTOK // _NW
_L = 20
_CH = 32
_NCH = _BPW // _CH
_NBUF = 5
_GD = 2


def _table_body(emb_ref, taskw_ref, w1at_ref, w1bt_ref, b1_ref, w2t_ref,
                b2_ref, out_ref):
    g = jnp.dot(emb_ref[...], w1at_ref[...],
                preferred_element_type=jnp.float32)
    hb = jnp.dot(taskw_ref[...], w1bt_ref[...],
                 preferred_element_type=jnp.float32) + b1_ref[...]
    inv_sqrt2 = 0.7071067811865476
    for li in range(2):
        a = g + hb[li, :][None, :]
        h = 0.5 * a * (1.0 + lax.erf(a * inv_sqrt2))
        out_ref[li] = jnp.dot(h, w2t_ref[...],
                              preferred_element_type=jnp.float32) + b2_ref[...]


def _build_table(emb_w, task_w, W1, b1, W2, b2):
    w1at = W1[:, :_MD].T
    w1bt = W1[:, _MD:].T
    w2t = W2.T
    b1r = b1.reshape(1, _H)
    b2r = b2.reshape(1, _MD)
    return pl.pallas_call(
        _table_body,
        grid=(_VOCAB // _RB,),
        in_specs=[
            pl.BlockSpec((_RB, _MD), lambda i: (i, 0)),
            pl.BlockSpec((2, _MD), lambda i: (0, 0)),
            pl.BlockSpec((_MD, _H), lambda i: (0, 0)),
            pl.BlockSpec((_MD, _H), lambda i: (0, 0)),
            pl.BlockSpec((1, _H), lambda i: (0, 0)),
            pl.BlockSpec((_H, _MD), lambda i: (0, 0)),
            pl.BlockSpec((1, _MD), lambda i: (0, 0)),
        ],
        out_specs=pl.BlockSpec((2, _RB, _MD), lambda i: (0, i, 0)),
        out_shape=jax.ShapeDtypeStruct((2, _VOCAB, _MD), jnp.float32),
    )(emb_w, task_w, w1at, w1bt, b1r, w2t, b2r)


def _sc_gather(table, t_flat, l_flat):
    mesh = plsc.VectorSubcoreMesh(core_axis_name="c", subcore_axis_name="s")

    @functools.partial(
        pl.kernel,
        out_type=jax.ShapeDtypeStruct((_NTOK, _MD), jnp.float32),
        mesh=mesh,
        scratch_types=[
            pltpu.VMEM((_BPW,), jnp.int32),
            pltpu.VMEM((_BPW,), jnp.int32),
            pltpu.VMEM((_NBUF, _CH, _MD), jnp.float32),
        ] + [pltpu.SemaphoreType.DMA] * (2 * _NBUF),
    )
    def k(table_hbm, t_hbm, l_hbm, out_hbm, idx_v, l_v, rows_v, *sems):
        gsems, wsems = sems[:_NBUF], sems[_NBUF:]
        wid = lax.axis_index("s") * _NC + lax.axis_index("c")
        base = pl.multiple_of(wid * _BPW, _BPW)
        pltpu.sync_copy(t_hbm.at[pl.ds(base, _BPW)], idx_v)
        pltpu.sync_copy(l_hbm.at[pl.ds(base, _BPW)], l_v)

        def ibody(i, carry):
            off = pl.multiple_of(i * _LANES, _LANES)
            idx_v[pl.ds(off, _LANES)] = (
                idx_v[pl.ds(off, _LANES)] + l_v[pl.ds(off, _LANES)] * _VOCAB)
            return carry
        lax.fori_loop(0, _BPW // _LANES, ibody, 0)

        def start_gather(c, s):
            pltpu.async_copy(
                table_hbm.at[idx_v.at[pl.ds(c * _CH, _CH)]],
                rows_v.at[s], gsems[s])

        def drain_gather(s):
            pltpu.make_async_copy(
                table_hbm.at[pl.ds(0, _CH)], rows_v.at[s], gsems[s]).wait()

        def start_write(c, s):
            pltpu.async_copy(rows_v.at[s],
                             out_hbm.at[pl.ds(base + c * _CH, _CH)],
                             wsems[s])

        def drain_write(s):
            pltpu.make_async_copy(rows_v.at[s],
                                  out_hbm.at[pl.ds(base, _CH)],
                                  wsems[s]).wait()

        for c in range(_GD):
            start_gather(c, c % _NBUF)

        def chunk_group(g, carry):
            for s in range(_NBUF):
                c = g * _NBUF + s
                s2 = (s + _GD) % _NBUF

                @pl.when(c + _GD >= _NBUF)
                def _():
                    drain_write(s2)

                @pl.when(c + _GD < _NCH)
                def _():
                    start_gather(c + _GD, s2)
                drain_gather(s)
                start_write(c, s)
            return carry
        lax.fori_loop(0, _NCH // _NBUF, chunk_group, 0)

        for i in range(_GD, _NBUF):
            drain_write((_NCH - _NBUF + i) % _NBUF)

    return k(table, t_flat, l_flat)


def kernel(t, l, emb_w, task_w, W1, b1, W2, b2):
    table = _build_table(emb_w, task_w, W1, b1, W2, b2)
    table_flat = table.reshape(2 * _VOCAB, _MD)
    tp = t.T.reshape(_NTOK).astype(jnp.int32)
    lp = l.T.reshape(_NTOK).astype(jnp.int32)
    out_sm = _sc_gather(table_flat, tp, lp)
    return out_sm.reshape(_L, _NTOK // _L, _MD).transpose(1, 0, 2)


# speedup vs baseline: 1.0345x; 1.0345x over previous
import functools

import jax
import jax.numpy as jnp
from jax import lax
from jax.experimental import pallas as pl
from jax.experimental.pallas import tpu as pltpu
from jax.experimental.pallas import tpu_sc as plsc

_VOCAB = 65536
_MD = 512
_H = 256
_NTOK = 16384 * 20

_RB = 2048

_NC, _NS, _LANES = 2, 16, 16
_NW = _NC * _NS
_BPW = _NTOK // _NW
_L = 20
_CH = 32
_NCH = _BPW // _CH
_NBUF = 5
_GD = 2


def _table_body(emb_ref, taskw_ref, w1at_ref, w1bt_ref, b1_ref, w2t_ref,
                b2_ref, out_ref):
    g = jnp.dot(emb_ref[...], w1at_ref[...],
                preferred_element_type=jnp.float32)
    hb = jnp.dot(taskw_ref[...], w1bt_ref[...],
                 preferred_element_type=jnp.float32) + b1_ref[...]
    inv_sqrt2 = 0.7071067811865476
    for li in range(2):
        a = g + hb[li, :][None, :]
        h = 0.5 * a * (1.0 + lax.erf(a * inv_sqrt2))
        out_ref[li] = jnp.dot(h, w2t_ref[...],
                              preferred_element_type=jnp.float32) + b2_ref[...]


def _build_table(emb_w, task_w, W1, b1, W2, b2):
    w1at = W1[:, :_MD].T
    w1bt = W1[:, _MD:].T
    w2t = W2.T
    b1r = b1.reshape(1, _H)
    b2r = b2.reshape(1, _MD)
    return pl.pallas_call(
        _table_body,
        grid=(_VOCAB // _RB,),
        in_specs=[
            pl.BlockSpec((_RB, _MD), lambda i: (i, 0)),
            pl.BlockSpec((2, _MD), lambda i: (0, 0)),
            pl.BlockSpec((_MD, _H), lambda i: (0, 0)),
            pl.BlockSpec((_MD, _H), lambda i: (0, 0)),
            pl.BlockSpec((1, _H), lambda i: (0, 0)),
            pl.BlockSpec((_H, _MD), lambda i: (0, 0)),
            pl.BlockSpec((1, _MD), lambda i: (0, 0)),
        ],
        out_specs=pl.BlockSpec((2, _RB, _MD), lambda i: (0, i, 0)),
        out_shape=jax.ShapeDtypeStruct((2, _VOCAB, _MD), jnp.float32),
    )(emb_w, task_w, w1at, w1bt, b1r, w2t, b2r)


def _sc_gather(table, t_flat, l_flat):
    mesh = plsc.VectorSubcoreMesh(core_axis_name="c", subcore_axis_name="s")

    @functools.partial(
        pl.kernel,
        out_type=jax.ShapeDtypeStruct((_NTOK, _MD), jnp.float32),
        mesh=mesh,
        scratch_types=[
            pltpu.VMEM((_BPW,), jnp.int32),
            pltpu.VMEM((_BPW,), jnp.int32),
            pltpu.VMEM((_NBUF, _CH, _MD), jnp.float32),
        ] + [pltpu.SemaphoreType.DMA] * (2 * _NBUF),
    )
    def k(table_hbm, t_hbm, l_hbm, out_hbm, idx_v, l_v, rows_v, *sems):
        gsems, wsems = sems[:_NBUF], sems[_NBUF:]
        wid = lax.axis_index("s") * _NC + lax.axis_index("c")
        base = pl.multiple_of(wid * _BPW, _BPW)
        pltpu.sync_copy(t_hbm.at[pl.ds(base, _BPW)], idx_v)
        pltpu.sync_copy(l_hbm.at[pl.ds(base, _BPW)], l_v)

        def ibody(i, carry):
            off = pl.multiple_of(i * _LANES, _LANES)
            idx_v[pl.ds(off, _LANES)] = (
                idx_v[pl.ds(off, _LANES)] + l_v[pl.ds(off, _LANES)] * _VOCAB)
            return carry
        lax.fori_loop(0, _BPW // _LANES, ibody, 0)

        def start_gather(c, s):
            pltpu.async_copy(
                table_hbm.at[idx_v.at[pl.ds(c * _CH, _CH)]],
                rows_v.at[s], gsems[s])

        def drain_gather(s):
            pltpu.make_async_copy(
                table_hbm.at[pl.ds(0, _CH)], rows_v.at[s], gsems[s]).wait()

        def start_write(c, s):
            pltpu.async_copy(rows_v.at[s],
                             out_hbm.at[pl.ds(base + c * _CH, _CH)],
                             wsems[s])

        def drain_write(s):
            pltpu.make_async_copy(rows_v.at[s],
                                  out_hbm.at[pl.ds(base, _CH)],
                                  wsems[s]).wait()

        for c in range(_GD):
            start_gather(c, c % _NBUF)

        def chunk_group(g, carry):
            for s in range(_NBUF):
                c = g * _NBUF + s
                s2 = (s + _GD) % _NBUF

                @pl.when(c + _GD >= _NBUF)
                def _():
                    drain_write(s2)

                @pl.when(c + _GD < _NCH)
                def _():
                    start_gather(c + _GD, s2)
                drain_gather(s)
                start_write(c, s)
            return carry
        lax.fori_loop(0, _NCH // _NBUF, chunk_group, 0)

        for i in range(_GD, _NBUF):
            drain_write((_NCH - _NBUF + i) % _NBUF)

    return k(table, t_flat, l_flat)


def kernel(t, l, emb_w, task_w, W1, b1, W2, b2):
    table = _build_table(emb_w, task_w, W1, b1, W2, b2)
    table_flat = table.reshape(2 * _VOCAB, _MD)
    tp = t.T.reshape(_NTOK).astype(jnp.int32)
    lp = l.T.reshape(_NTOK).astype(jnp.int32)
    out_sm = _sc_gather(table_flat, tp, lp)
    return out_sm.reshape(_L, _NTOK // _L, _MD).transpose(1, 0, 2)
